# baseline (speedup 1.0000x reference)
import functools

import jax
import jax.numpy as jnp
from jax.experimental import pallas as pl

NUM_LEVELS = 4
K = 1024
D = 256
BLOCK_B = 512


def _rq_kernel(x_ref, cb_ref, cbn_ref, codes_ref, quant_ref):
    residual = x_ref[...]
    bb = residual.shape[0]
    quant = jnp.zeros_like(residual)
    iota_k = jax.lax.broadcasted_iota(jnp.int32, (bb, K), 1)
    for l in range(NUM_LEVELS):
        cb = cb_ref[l]
        r_norm = jnp.sum(residual * residual, axis=1, keepdims=True)
        d2 = (
            r_norm
            - 2.0 * jax.lax.dot_general(
                residual, cb,
                dimension_numbers=(((1,), (1,)), ((), ())),
                preferred_element_type=jnp.float32,
            )
        ) + cbn_ref[l][None, :]
        m = jnp.min(d2, axis=1, keepdims=True)
        codes = jnp.min(jnp.where(d2 == m, iota_k, K), axis=1)
        onehot = (iota_k == codes[:, None]).astype(jnp.float32)
        q = jax.lax.dot_general(
            onehot, cb,
            dimension_numbers=(((1,), (0,)), ((), ())),
            preferred_element_type=jnp.float32,
            precision=jax.lax.Precision.HIGHEST,
        )
        codes_ref[l, :] = codes.astype(jnp.int32)
        quant = quant + q
        residual = residual - q
    quant_ref[...] = quant


@jax.jit
def kernel(x, codebooks):
    b, d = x.shape
    cb_norms = jnp.stack(
        [jnp.sum(codebooks[l] * codebooks[l], axis=1)
         for l in range(NUM_LEVELS)], axis=0)
    grid = (b // BLOCK_B,)
    codes_t, quant = pl.pallas_call(
        _rq_kernel,
        grid=grid,
        in_specs=[
            pl.BlockSpec((BLOCK_B, d), lambda i: (i, 0)),
            pl.BlockSpec((NUM_LEVELS, K, D), lambda i: (0, 0, 0)),
            pl.BlockSpec((NUM_LEVELS, K), lambda i: (0, 0)),
        ],
        out_specs=[
            pl.BlockSpec((NUM_LEVELS, BLOCK_B), lambda i: (0, i)),
            pl.BlockSpec((BLOCK_B, d), lambda i: (i, 0)),
        ],
        out_shape=[
            jax.ShapeDtypeStruct((NUM_LEVELS, b), jnp.int32),
            jax.ShapeDtypeStruct((b, d), jnp.float32),
        ],
    )(x, codebooks, cb_norms)
    return codes_t.T, quant


# speedup vs baseline: 1.5892x; 1.5892x over previous
import functools

import jax
import jax.numpy as jnp
from jax.experimental import pallas as pl

NUM_LEVELS = 4
K = 1024
D = 256
BLOCK_B = 512


def _rq_kernel(x_ref, cb_ref, cbhi_ref, cbmid_ref, cblo_ref, cbn_ref,
               codes_ref, quant_ref):
    residual = x_ref[...]
    bb = residual.shape[0]
    quant = jnp.zeros_like(residual)
    iota_k = jax.lax.broadcasted_iota(jnp.int32, (bb, K), 1)
    for l in range(NUM_LEVELS):
        cb = cb_ref[l]
        r_norm = jnp.sum(residual * residual, axis=1, keepdims=True)
        d2 = (
            r_norm
            - 2.0 * jax.lax.dot_general(
                residual, cb,
                dimension_numbers=(((1,), (1,)), ((), ())),
                preferred_element_type=jnp.float32,
            )
        ) + cbn_ref[l][None, :]
        m = jnp.min(d2, axis=1, keepdims=True)
        codes = jnp.min(jnp.where(d2 == m, iota_k, K), axis=1)
        onehot = (iota_k == codes[:, None]).astype(jnp.bfloat16)
        q = jnp.float32(0)
        for limb_ref in (cbhi_ref, cbmid_ref, cblo_ref):
            q = q + jax.lax.dot_general(
                onehot, limb_ref[l],
                dimension_numbers=(((1,), (0,)), ((), ())),
                preferred_element_type=jnp.float32,
            )
        codes_ref[l, :] = codes.astype(jnp.int32)
        quant = quant + q
        residual = residual - q
    quant_ref[...] = quant


@jax.jit
def kernel(x, codebooks):
    b, d = x.shape
    cb_norms = jnp.stack(
        [jnp.sum(codebooks[l] * codebooks[l], axis=1)
         for l in range(NUM_LEVELS)], axis=0)
    cb_hi = codebooks.astype(jnp.bfloat16)
    r1 = codebooks - cb_hi.astype(jnp.float32)
    cb_mid = r1.astype(jnp.bfloat16)
    cb_lo = (r1 - cb_mid.astype(jnp.float32)).astype(jnp.bfloat16)
    grid = (b // BLOCK_B,)
    cb_full_spec = pl.BlockSpec((NUM_LEVELS, K, D), lambda i: (0, 0, 0))
    codes_t, quant = pl.pallas_call(
        _rq_kernel,
        grid=grid,
        in_specs=[
            pl.BlockSpec((BLOCK_B, d), lambda i: (i, 0)),
            cb_full_spec,
            cb_full_spec,
            cb_full_spec,
            cb_full_spec,
            pl.BlockSpec((NUM_LEVELS, K), lambda i: (0, 0)),
        ],
        out_specs=[
            pl.BlockSpec((NUM_LEVELS, BLOCK_B), lambda i: (0, i)),
            pl.BlockSpec((BLOCK_B, d), lambda i: (i, 0)),
        ],
        out_shape=[
            jax.ShapeDtypeStruct((NUM_LEVELS, b), jnp.int32),
            jax.ShapeDtypeStruct((b, d), jnp.float32),
        ],
    )(x, codebooks, cb_hi, cb_mid, cb_lo, cb_norms)
    return codes_t.T, quant


# speedup vs baseline: 1.9545x; 1.2298x over previous
import functools

import jax
import jax.numpy as jnp
from jax.experimental import pallas as pl

NUM_LEVELS = 4
K = 1024
D = 256
BLOCK_B = 512


def _rq_kernel(x_ref, cb_ref, b0_ref, b1_ref, b2_ref, b3_ref, cbn_ref,
               codes_ref, quant_ref):
    residual = x_ref[...]
    bb = residual.shape[0]
    quant = jnp.zeros_like(residual)
    iota_k = jax.lax.broadcasted_iota(jnp.int32, (bb, K), 1)
    for l in range(NUM_LEVELS):
        cb = cb_ref[l]
        r_norm = jnp.sum(residual * residual, axis=1, keepdims=True)
        d2 = (
            r_norm
            - 2.0 * jax.lax.dot_general(
                residual, cb,
                dimension_numbers=(((1,), (1,)), ((), ())),
                preferred_element_type=jnp.float32,
            )
        ) + cbn_ref[l][None, :]
        m = jnp.min(d2, axis=1, keepdims=True)
        codes = jnp.min(jnp.where(d2 == m, iota_k, K), axis=1)
        onehot_i8 = (iota_k == codes[:, None]).astype(jnp.int8)
        dn = (((1,), (0,)), ((), ()))
        parts = []
        for bref in (b0_ref, b1_ref, b2_ref, b3_ref):
            v = jax.lax.dot_general(onehot_i8, bref[l], dn,
                                    preferred_element_type=jnp.int32)
            parts.append(jnp.bitwise_and(v, 0xFF))
        bits = (parts[0] | (parts[1] << 8) | (parts[2] << 16)
                | (parts[3] << 24))
        q = jax.lax.bitcast_convert_type(bits, jnp.float32)
        codes_ref[l, :] = codes.astype(jnp.int32)
        quant = quant + q
        residual = residual - q
    quant_ref[...] = quant


@jax.jit
def kernel(x, codebooks):
    b, d = x.shape
    cb_norms = jnp.stack(
        [jnp.sum(codebooks[l] * codebooks[l], axis=1)
         for l in range(NUM_LEVELS)], axis=0)
    cb_bytes = jax.lax.bitcast_convert_type(codebooks, jnp.int8)
    byte_planes = [cb_bytes[..., j] for j in range(4)]
    grid = (b // BLOCK_B,)
    cb_spec = pl.BlockSpec((NUM_LEVELS, K, D), lambda i: (0, 0, 0))
    codes_t, quant = pl.pallas_call(
        _rq_kernel,
        grid=grid,
        in_specs=[
            pl.BlockSpec((BLOCK_B, d), lambda i: (i, 0)),
            cb_spec, cb_spec, cb_spec, cb_spec, cb_spec,
            pl.BlockSpec((NUM_LEVELS, K), lambda i: (0, 0)),
        ],
        out_specs=[
            pl.BlockSpec((NUM_LEVELS, BLOCK_B), lambda i: (0, i)),
            pl.BlockSpec((BLOCK_B, d), lambda i: (i, 0)),
        ],
        out_shape=[
            jax.ShapeDtypeStruct((NUM_LEVELS, b), jnp.int32),
            jax.ShapeDtypeStruct((b, d), jnp.float32),
        ],
    )(x, codebooks, *byte_planes, cb_norms)
    return codes_t.T, quant


# speedup vs baseline: 2.0208x; 1.0339x over previous
import functools

import jax
import jax.numpy as jnp
from jax.experimental import pallas as pl

NUM_LEVELS = 4
K = 1024
D = 256
BLOCK_B = 512


def _rq_kernel(x_ref, cb2_ref, hi_ref, b0_ref, b1_ref, cbn_ref,
               codes_ref, quant_ref):
    residual = x_ref[...]
    bb = residual.shape[0]
    quant = jnp.zeros_like(residual)
    iota_k = jax.lax.broadcasted_iota(jnp.int32, (bb, K), 1)
    dn_t = (((1,), (1,)), ((), ()))
    dn = (((1,), (0,)), ((), ()))
    for l in range(NUM_LEVELS):
        r_norm = jnp.sum(residual * residual, axis=1, keepdims=True)
        d2 = (
            r_norm
            - jax.lax.dot_general(
                residual, cb2_ref[l], dn_t,
                preferred_element_type=jnp.float32,
            )
        ) + cbn_ref[l][None, :]
        codes = jnp.argmin(d2, axis=1).astype(jnp.int32)
        onehot_bf = (iota_k == codes[:, None]).astype(jnp.bfloat16)
        q_hi = jax.lax.dot_general(onehot_bf, hi_ref[l], dn,
                                   preferred_element_type=jnp.float32)
        hi_bits = jax.lax.bitcast_convert_type(q_hi, jnp.int32)
        v0 = jax.lax.dot_general(onehot_bf, b0_ref[l], dn,
                                 preferred_element_type=jnp.float32)
        v1 = jax.lax.dot_general(onehot_bf, b1_ref[l], dn,
                                 preferred_element_type=jnp.float32)
        low16 = (v0 + v1).astype(jnp.int32)
        bits = hi_bits | low16
        q = jax.lax.bitcast_convert_type(bits, jnp.float32)
        codes_ref[l, :] = codes
        quant = quant + q
        residual = residual - q
    quant_ref[...] = quant


@jax.jit
def kernel(x, codebooks):
    b, d = x.shape
    cb_norms = jnp.stack(
        [jnp.sum(codebooks[l] * codebooks[l], axis=1)
         for l in range(NUM_LEVELS)], axis=0)
    cb_bytes = jax.lax.bitcast_convert_type(codebooks, jnp.uint8)
    b0_plane = cb_bytes[..., 0].astype(jnp.bfloat16)
    b1_plane = (cb_bytes[..., 1].astype(jnp.float32)
                * 256.0).astype(jnp.bfloat16)
    hi_plane = jax.lax.bitcast_convert_type(
        jax.lax.bitcast_convert_type(codebooks, jnp.int16)[..., 1],
        jnp.bfloat16)
    grid = (b // BLOCK_B,)
    cb_spec = pl.BlockSpec((NUM_LEVELS, K, D), lambda i: (0, 0, 0))
    codes_t, quant = pl.pallas_call(
        _rq_kernel,
        grid=grid,
        in_specs=[
            pl.BlockSpec((BLOCK_B, d), lambda i: (i, 0)),
            cb_spec, cb_spec, cb_spec, cb_spec,
            pl.BlockSpec((NUM_LEVELS, K), lambda i: (0, 0)),
        ],
        out_specs=[
            pl.BlockSpec((NUM_LEVELS, BLOCK_B), lambda i: (0, i)),
            pl.BlockSpec((BLOCK_B, d), lambda i: (i, 0)),
        ],
        out_shape=[
            jax.ShapeDtypeStruct((NUM_LEVELS, b), jnp.int32),
            jax.ShapeDtypeStruct((b, d), jnp.float32),
        ],
    )(x, 2.0 * codebooks, hi_plane, b0_plane, b1_plane, cb_norms)
    return codes_t.T, quant
